# baseline (speedup 1.0000x reference)
import functools

import jax
import jax.numpy as jnp
from jax import lax
from jax.experimental import pallas as pl
from jax.experimental.pallas import tpu as pltpu
from jax.experimental.pallas import tpu_sc as plsc

NC = 2
NS = 16
NW = NC * NS

B = 16384 * 50
D = 32
BPW = B // NW

G = 1280
GP_SG = 1
SG_ROWS = G * GP_SG
NSG = BPW // SG_ROWS
NBUF = 2

_mesh = plsc.VectorSubcoreMesh(core_axis_name="c", subcore_axis_name="s")


@functools.partial(
    pl.kernel,
    mesh=_mesh,
    compiler_params=pltpu.CompilerParams(use_tc_tiling_on_sc=False),
    out_type=jax.ShapeDtypeStruct((B, D), jnp.float32),
    scratch_types=[
        pltpu.VMEM((BPW,), jnp.int32),
        pltpu.VMEM((SG_ROWS, D), jnp.float32),
        pltpu.VMEM((SG_ROWS, D), jnp.float32),
        pltpu.SemaphoreType.DMA,
        pltpu.SemaphoreType.DMA,
        pltpu.SemaphoreType.DMA,
        pltpu.SemaphoreType.DMA,
    ],
)
def _embed(idx_hbm, tbl_hbm, out_hbm, idx_v, buf0, buf1,
           gsem0, gsem1, osem0, osem1):
    wid = lax.axis_index("s") * NC + lax.axis_index("c")
    base = wid * BPW
    bufs = (buf0, buf1)
    gsems = (gsem0, gsem1)
    osems = (osem0, osem1)

    pltpu.sync_copy(idx_hbm.at[pl.ds(base, BPW)], idx_v)

    def issue_gathers(sg, b):
        for j in range(GP_SG):
            pltpu.async_copy(
                tbl_hbm.at[idx_v.at[pl.ds(sg * SG_ROWS + j * G, G)]],
                bufs[b].at[pl.ds(j * G, G), :],
                gsems[b],
            )

    def drain_gathers(b):
        pltpu.make_async_copy(
            out_hbm.at[pl.ds(base, SG_ROWS), :], bufs[b], gsems[b]
        ).wait()

    def start_out(b, sg):
        pltpu.async_copy(
            bufs[b], out_hbm.at[pl.ds(base + sg * SG_ROWS, SG_ROWS), :],
            osems[b],
        )

    def wait_out(b):
        pltpu.make_async_copy(
            bufs[b], out_hbm.at[pl.ds(base, SG_ROWS), :], osems[b]
        ).wait()

    for b in range(NBUF):
        issue_gathers(b, b)

    def body(k, carry):
        for b in range(NBUF):
            sg = k * NBUF + b
            drain_gathers(b)
            start_out(b, sg)
            wait_out(b)
            issue_gathers(sg + NBUF, b)
        return carry

    lax.fori_loop(0, NSG // NBUF - 1, body, 0)

    for b in range(NBUF):
        drain_gathers(b)
        start_out(b, NSG - NBUF + b)
    for b in range(NBUF):
        wait_out(b)


def kernel(x, weight):
    idx = x.reshape(-1).astype(jnp.int32)
    out = _embed(idx, weight)
    return out.reshape(x.shape + (weight.shape[1],))


# speedup vs baseline: 1.6234x; 1.6234x over previous
import functools

import jax
import jax.numpy as jnp
from jax import lax
from jax.experimental import pallas as pl
from jax.experimental.pallas import tpu as pltpu
from jax.experimental.pallas import tpu_sc as plsc

NC = 2
NS = 16
NW = NC * NS

NROW = 16384
RL = 50
D = 32
RPW = NROW // NW

NI = 16
NSG = RPW // NI
NBUF = 2

_mesh = plsc.VectorSubcoreMesh(core_axis_name="c", subcore_axis_name="s")


@functools.partial(
    pl.kernel,
    mesh=_mesh,
    compiler_params=pltpu.CompilerParams(use_tc_tiling_on_sc=False),
    out_type=jax.ShapeDtypeStruct((NROW, RL, D), jnp.float32),
    scratch_types=[
        pltpu.VMEM((RPW, RL), jnp.int32),
        pltpu.VMEM((NI, RL, D), jnp.float32),
        pltpu.VMEM((NI, RL, D), jnp.float32),
        pltpu.SemaphoreType.DMA,
        pltpu.SemaphoreType.DMA,
        pltpu.SemaphoreType.DMA,
        pltpu.SemaphoreType.DMA,
    ],
)
def _embed(idx_hbm, tbl_hbm, out_hbm, idx_v, buf0, buf1,
           gsem0, gsem1, osem0, osem1):
    wid = lax.axis_index("s") * NC + lax.axis_index("c")
    base = wid * RPW
    bufs = (buf0, buf1)
    gsems = (gsem0, gsem1)
    osems = (osem0, osem1)

    pltpu.sync_copy(idx_hbm.at[pl.ds(base, RPW), :], idx_v)

    def issue_gathers(sg, b):
        for k in range(NI):
            pltpu.async_copy(
                tbl_hbm.at[idx_v.at[sg * NI + k]],
                bufs[b].at[k],
                gsems[b],
            )

    def drain_gathers(b):
        pltpu.make_async_copy(
            out_hbm.at[pl.ds(base, NI), :, :], bufs[b], gsems[b]
        ).wait()

    def start_out(b, sg):
        pltpu.async_copy(
            bufs[b], out_hbm.at[pl.ds(base + sg * NI, NI), :, :],
            osems[b],
        )

    def wait_out(b):
        pltpu.make_async_copy(
            bufs[b], out_hbm.at[pl.ds(base, NI), :, :], osems[b]
        ).wait()

    for b in range(NBUF):
        issue_gathers(b, b)

    def body(k, carry):
        for b in range(NBUF):
            sg = k * NBUF + b
            drain_gathers(b)
            start_out(b, sg)
            wait_out(b)
            issue_gathers(sg + NBUF, b)
        return carry

    lax.fori_loop(0, NSG // NBUF - 1, body, 0)

    for b in range(NBUF):
        drain_gathers(b)
        start_out(b, NSG - NBUF + b)
    for b in range(NBUF):
        wait_out(b)


def kernel(x, weight):
    return _embed(x.astype(jnp.int32), weight)
